# baseline (speedup 1.0000x reference)
import functools

import jax
import jax.numpy as jnp
from jax import lax
from jax.experimental import pallas as pl
from jax.experimental.pallas import tpu as pltpu
from jax.experimental.pallas import tpu_sc as plsc

N = 10000
E = 320000
D = 128
DOUT = 40

NW = 32
EPW = E // NW
CHUNK = 125
NCH = EPW // CHUNK
HNCH = NCH // 2
OWN = 640
CC = 80
HW = 32


@functools.cache
def _get_deg_kernel():
    mesh = plsc.VectorSubcoreMesh(core_axis_name="c", subcore_axis_name="s")
    return functools.partial(
        pl.kernel,
        mesh=mesh,
        out_type=jax.ShapeDtypeStruct((2 * N, HW), jnp.float32),
        scratch_types=[
            pltpu.VMEM_SHARED((N, HW), jnp.float32),
            pltpu.VMEM((NCH, CHUNK), jnp.int32),
            pltpu.VMEM((CHUNK, HW), jnp.float32),
            pltpu.VMEM((CC, HW), jnp.float32),
            pltpu.SemaphoreType.DMA,
        ],
    )(_deg_body)


def _deg_body(dst_hbm, e1_hbm, z16_hbm, out_hbm, hist, dst_v, ones_v, stage_v, sem):
    cid = lax.axis_index("c")
    sid = lax.axis_index("s")
    wid = sid * 2 + cid

    pltpu.sync_copy(dst_hbm.at[pl.ds(wid * NCH, NCH)], dst_v)
    pltpu.sync_copy(e1_hbm, ones_v)
    pltpu.sync_copy(z16_hbm, stage_v)

    nrows = jnp.where(sid < 15, OWN, N - 15 * OWN)
    trips = nrows // CC

    def zbody(r, carry):
        pltpu.sync_copy(stage_v, hist.at[pl.ds(sid * OWN + r * CC, CC)])
        return carry
    lax.fori_loop(0, trips, zbody, 0)
    plsc.subcore_barrier()

    def body(c, carry):
        pltpu.sync_copy(ones_v, hist.at[dst_v.at[c]], add=True)
        return carry
    lax.fori_loop(0, NCH, body, 0)
    plsc.subcore_barrier()

    def obody(r, carry):
        base = sid * OWN + r * CC
        pltpu.sync_copy(hist.at[pl.ds(base, CC)], stage_v)
        pltpu.sync_copy(stage_v, out_hbm.at[pl.ds(cid * N + base, CC)])
        return carry
    lax.fori_loop(0, trips, obody, 0)


@functools.cache
def _get_spmm_kernel():
    mesh = plsc.VectorSubcoreMesh(core_axis_name="c", subcore_axis_name="s")
    return functools.partial(
        pl.kernel,
        mesh=mesh,
        out_type=jax.ShapeDtypeStruct((2 * N, D), jnp.float32),
        scratch_types=[
            pltpu.VMEM_SHARED((N, D), jnp.float32),
            pltpu.VMEM((HNCH, CHUNK), jnp.int32),
            pltpu.VMEM((HNCH, CHUNK), jnp.int32),
            pltpu.VMEM((CHUNK, D), jnp.float32),
            pltpu.VMEM((CHUNK, D), jnp.float32),
            pltpu.SemaphoreType.DMA,
            pltpu.SemaphoreType.DMA,
        ],
    )(_spmm_body)


def _spmm_body(g_hbm, src_hbm, dst_hbm, zrow_hbm, out_hbm,
               acc, src_v, dst_v, rows_a, rows_b, sem_a, sem_b):
    cid = lax.axis_index("c")
    sid = lax.axis_index("s")
    wid = sid * 2 + cid

    pltpu.sync_copy(zrow_hbm, rows_a.at[pl.ds(0, CC)])

    nrows = jnp.where(sid < 15, OWN, N - 15 * OWN)
    trips = nrows // CC

    def zbody(r, carry):
        pltpu.sync_copy(rows_a.at[pl.ds(0, CC)],
                        acc.at[pl.ds(sid * OWN + r * CC, CC)])
        return carry
    lax.fori_loop(0, trips, zbody, 0)
    plsc.subcore_barrier()

    for h in range(2):
        base = wid * NCH + h * HNCH
        pltpu.sync_copy(src_hbm.at[pl.ds(base, HNCH)], src_v)
        pltpu.sync_copy(dst_hbm.at[pl.ds(base, HNCH)], dst_v)
        pltpu.async_copy(g_hbm.at[src_v.at[0]], rows_a, sem_a)

        def body(i, carry):
            cl = i * 2

            pltpu.async_copy(g_hbm.at[src_v.at[cl + 1]], rows_b, sem_b)
            pltpu.make_async_copy(g_hbm.at[src_v.at[cl]], rows_a, sem_a).wait()
            pltpu.sync_copy(rows_a, acc.at[dst_v.at[cl]], add=True)

            @pl.when(cl + 2 < HNCH)
            def _prefetch_a():
                pltpu.async_copy(g_hbm.at[src_v.at[cl + 2]], rows_a, sem_a)

            pltpu.make_async_copy(g_hbm.at[src_v.at[cl + 1]], rows_b, sem_b).wait()
            pltpu.sync_copy(rows_b, acc.at[dst_v.at[cl + 1]], add=True)
            return carry

        lax.fori_loop(0, HNCH // 2, body, 0)
    plsc.subcore_barrier()

    def obody(r, carry):
        base = sid * OWN + r * CC
        pltpu.sync_copy(acc.at[pl.ds(base, CC)], rows_a.at[pl.ds(0, CC)])
        pltpu.sync_copy(rows_a.at[pl.ds(0, CC)],
                        out_hbm.at[pl.ds(cid * N + base, CC)])
        return carry
    lax.fori_loop(0, trips, obody, 0)


_RB = 1000
_GRID = N // _RB


def _dis_block(ha, hb):
    deg = ha[:, 0:1] + hb[:, 0:1] + 1.0
    return lax.rsqrt(deg)


def _k2_body(x_ref, w1_ref, ha_ref, hb_ref, g1_ref):
    dis = _dis_block(ha_ref[...], hb_ref[...])
    h = jnp.dot(x_ref[...], w1_ref[...], preferred_element_type=jnp.float32)
    g1_ref[...] = dis * h


def _k4_body(sa_ref, sb_ref, g1_ref, ha_ref, hb_ref, b1_ref, w2_ref,
             g2_ref, m1_ref):
    i = pl.program_id(0)
    dis = _dis_block(ha_ref[...], hb_ref[...])
    x1 = jnp.maximum(dis * (sa_ref[...] + sb_ref[...] + g1_ref[...])
                     + b1_ref[...], 0.0)
    colsum = jnp.sum(x1, axis=0, keepdims=True)

    @pl.when(i == 0)
    def _():
        m1_ref[...] = colsum

    @pl.when(i > 0)
    def _():
        m1_ref[...] += colsum

    h2 = jnp.dot(x1, w2_ref[...], preferred_element_type=jnp.float32)
    g2_ref[...] = dis * h2


def _k6_body(sa_ref, sb_ref, g2_ref, ha_ref, hb_ref, b2_ref, m1_ref,
             linw_ref, linb_ref, fcw_ref, fcb_ref, out_ref, m2_acc):
    i = pl.program_id(0)
    dis = _dis_block(ha_ref[...], hb_ref[...])
    x2 = jnp.maximum(dis * (sa_ref[...] + sb_ref[...] + g2_ref[...])
                     + b2_ref[...], 0.0)
    colsum = jnp.sum(x2, axis=0, keepdims=True)

    @pl.when(i == 0)
    def _():
        m2_acc[...] = colsum

    @pl.when(i > 0)
    def _():
        m2_acc[...] += colsum

    @pl.when(i == _GRID - 1)
    def _():
        inv_n = 1.0 / N
        m1 = m1_ref[...] * inv_n
        m2 = m2_acc[...] * inv_n
        pooled = (jnp.dot(m1, linw_ref[0:D, :], preferred_element_type=jnp.float32)
                  + jnp.dot(m2, linw_ref[D:2 * D, :], preferred_element_type=jnp.float32)
                  + linb_ref[...])
        logits = jnp.dot(pooled, fcw_ref[...], preferred_element_type=jnp.float32) \
            + fcb_ref[...]
        zmax = jnp.max(logits, axis=1, keepdims=True)
        ez = jnp.exp(logits - zmax)
        out_ref[...] = ez / jnp.sum(ez, axis=1, keepdims=True)


def kernel(x, edge_index, W1, b1, W2, b2, lin_W, lin_b, fc_W, fc_b):
    src = edge_index[0].astype(jnp.int32).reshape(NW * NCH, CHUNK)
    dst = edge_index[1].astype(jnp.int32).reshape(NW * NCH, CHUNK)
    e1 = jnp.zeros((CHUNK, HW), jnp.float32).at[:, 0].set(1.0)
    z16 = jnp.zeros((CC, HW), jnp.float32)
    zrow = jnp.zeros((CC, D), jnp.float32)
    b1r = b1.reshape(1, D)
    b2r = b2.reshape(1, D)
    linbr = lin_b.reshape(1, D)
    fcbr = fc_b.reshape(1, DOUT)

    hist = _get_deg_kernel()(dst, e1, z16)

    top = lambda i: (i, 0)
    bot = lambda i: (i + _GRID, 0)
    row_t = pl.BlockSpec((_RB, D), top)
    row_b = pl.BlockSpec((_RB, D), bot)
    h_spec_t = pl.BlockSpec((_RB, HW), top)
    h_spec_b = pl.BlockSpec((_RB, HW), bot)
    w_full = pl.BlockSpec((D, D), lambda i: (0, 0))
    vec_full = pl.BlockSpec((1, D), lambda i: (0, 0))

    g1 = pl.pallas_call(
        _k2_body,
        grid=(_GRID,),
        in_specs=[row_t, w_full, h_spec_t, h_spec_b],
        out_specs=row_t,
        out_shape=jax.ShapeDtypeStruct((N, D), jnp.float32),
    )(x, W1, hist, hist)

    s1 = _get_spmm_kernel()(g1, src, dst, zrow)

    g2, m1sum = pl.pallas_call(
        _k4_body,
        grid=(_GRID,),
        in_specs=[row_t, row_b, row_t, h_spec_t, h_spec_b, vec_full, w_full],
        out_specs=[row_t, vec_full],
        out_shape=[jax.ShapeDtypeStruct((N, D), jnp.float32),
                   jax.ShapeDtypeStruct((1, D), jnp.float32)],
    )(s1, s1, g1, hist, hist, b1r, W2)

    s2 = _get_spmm_kernel()(g2, src, dst, zrow)

    out = pl.pallas_call(
        _k6_body,
        grid=(_GRID,),
        in_specs=[row_t, row_b, row_t, h_spec_t, h_spec_b, vec_full, vec_full,
                  pl.BlockSpec((2 * D, D), lambda i: (0, 0)),
                  vec_full,
                  pl.BlockSpec((D, DOUT), lambda i: (0, 0)),
                  pl.BlockSpec((1, DOUT), lambda i: (0, 0))],
        out_specs=pl.BlockSpec((1, DOUT), lambda i: (0, 0)),
        out_shape=jax.ShapeDtypeStruct((1, DOUT), jnp.float32),
        scratch_shapes=[pltpu.VMEM((1, D), jnp.float32)],
    )(s2, s2, g2, hist, hist, b2r, m1sum, lin_W, linbr, fc_W, fcbr)

    return out


# speedup vs baseline: 1.3525x; 1.3525x over previous
import functools

import jax
import jax.numpy as jnp
from jax import lax
from jax.experimental import pallas as pl
from jax.experimental.pallas import tpu as pltpu
from jax.experimental.pallas import tpu_sc as plsc

N = 10000
E = 320000
D = 128
DOUT = 40

NW = 32
EPW = E // NW
CHUNK = 125
NCH = EPW // CHUNK
HNCH = NCH // 2
OWN = 640
CC = 80
HW = 128


@functools.cache
def _get_deg_kernel():
    mesh = plsc.VectorSubcoreMesh(core_axis_name="c", subcore_axis_name="s")
    return functools.partial(
        pl.kernel,
        mesh=mesh,
        out_type=jax.ShapeDtypeStruct((2 * N, HW), jnp.float32),
        scratch_types=[
            pltpu.VMEM_SHARED((N, HW), jnp.float32),
            pltpu.VMEM((NCH, CHUNK), jnp.int32),
            pltpu.VMEM((CHUNK, HW), jnp.float32),
            pltpu.VMEM((CC, HW), jnp.float32),
            pltpu.SemaphoreType.DMA,
        ],
    )(_deg_body)


def _deg_body(dst_hbm, e1_hbm, z16_hbm, out_hbm, hist, dst_v, ones_v, stage_v, sem):
    cid = lax.axis_index("c")
    sid = lax.axis_index("s")
    wid = sid * 2 + cid

    pltpu.sync_copy(dst_hbm.at[pl.ds(wid * NCH, NCH)], dst_v)
    pltpu.sync_copy(e1_hbm, ones_v)
    pltpu.sync_copy(z16_hbm, stage_v)

    nrows = jnp.where(sid < 15, OWN, N - 15 * OWN)
    trips = nrows // CC

    def zbody(r, carry):
        pltpu.sync_copy(stage_v, hist.at[pl.ds(sid * OWN + r * CC, CC)])
        return carry
    lax.fori_loop(0, trips, zbody, 0)
    plsc.subcore_barrier()

    def body(c, carry):
        pltpu.sync_copy(ones_v, hist.at[dst_v.at[c]], add=True)
        return carry
    lax.fori_loop(0, NCH, body, 0)
    plsc.subcore_barrier()

    def obody(r, carry):
        base = sid * OWN + r * CC
        pltpu.sync_copy(hist.at[pl.ds(base, CC)], stage_v)
        pltpu.sync_copy(stage_v, out_hbm.at[pl.ds(cid * N + base, CC)])
        return carry
    lax.fori_loop(0, trips, obody, 0)


@functools.cache
def _get_spmm_kernel():
    mesh = plsc.VectorSubcoreMesh(core_axis_name="c", subcore_axis_name="s")
    return functools.partial(
        pl.kernel,
        mesh=mesh,
        out_type=jax.ShapeDtypeStruct((2 * N, D), jnp.float32),
        scratch_types=[
            pltpu.VMEM_SHARED((N, D), jnp.float32),
            pltpu.VMEM((HNCH, CHUNK), jnp.int32),
            pltpu.VMEM((HNCH, CHUNK), jnp.int32),
            pltpu.VMEM((CHUNK, D), jnp.float32),
            pltpu.VMEM((CHUNK, D), jnp.float32),
            pltpu.SemaphoreType.DMA,
            pltpu.SemaphoreType.DMA,
        ],
    )(_spmm_body)


def _spmm_body(g_hbm, src_hbm, dst_hbm, zrow_hbm, out_hbm,
               acc, src_v, dst_v, rows_a, rows_b, sem_a, sem_b):
    cid = lax.axis_index("c")
    sid = lax.axis_index("s")
    wid = sid * 2 + cid

    pltpu.sync_copy(zrow_hbm, rows_a.at[pl.ds(0, CC)])

    nrows = jnp.where(sid < 15, OWN, N - 15 * OWN)
    trips = nrows // CC

    def zbody(r, carry):
        pltpu.sync_copy(rows_a.at[pl.ds(0, CC)],
                        acc.at[pl.ds(sid * OWN + r * CC, CC)])
        return carry
    lax.fori_loop(0, trips, zbody, 0)
    plsc.subcore_barrier()

    for h in range(2):
        base = wid * NCH + h * HNCH
        pltpu.sync_copy(src_hbm.at[pl.ds(base, HNCH)], src_v)
        pltpu.sync_copy(dst_hbm.at[pl.ds(base, HNCH)], dst_v)
        pltpu.async_copy(g_hbm.at[src_v.at[0]], rows_a, sem_a)

        def body(i, carry):
            cl = i * 2

            pltpu.async_copy(g_hbm.at[src_v.at[cl + 1]], rows_b, sem_b)
            pltpu.make_async_copy(g_hbm.at[src_v.at[cl]], rows_a, sem_a).wait()
            pltpu.sync_copy(rows_a, acc.at[dst_v.at[cl]], add=True)

            @pl.when(cl + 2 < HNCH)
            def _prefetch_a():
                pltpu.async_copy(g_hbm.at[src_v.at[cl + 2]], rows_a, sem_a)

            pltpu.make_async_copy(g_hbm.at[src_v.at[cl + 1]], rows_b, sem_b).wait()
            pltpu.sync_copy(rows_b, acc.at[dst_v.at[cl + 1]], add=True)
            return carry

        lax.fori_loop(0, HNCH // 2, body, 0)
    plsc.subcore_barrier()

    def obody(r, carry):
        base = sid * OWN + r * CC
        pltpu.sync_copy(acc.at[pl.ds(base, CC)], rows_a.at[pl.ds(0, CC)])
        pltpu.sync_copy(rows_a.at[pl.ds(0, CC)],
                        out_hbm.at[pl.ds(cid * N + base, CC)])
        return carry
    lax.fori_loop(0, trips, obody, 0)


_RB = 1000
_GRID = N // _RB


def _dis_block(ha, hb):
    deg = ha[:, 0:1] + hb[:, 0:1] + 1.0
    return lax.rsqrt(deg)


def _k2_body(x_ref, w1_ref, ha_ref, hb_ref, g1_ref):
    dis = _dis_block(ha_ref[...], hb_ref[...])
    h = jnp.dot(x_ref[...], w1_ref[...], preferred_element_type=jnp.float32)
    g1_ref[...] = dis * h


def _k4_body(sa_ref, sb_ref, g1_ref, ha_ref, hb_ref, b1_ref, w2_ref,
             g2_ref, m1_ref):
    i = pl.program_id(0)
    dis = _dis_block(ha_ref[...], hb_ref[...])
    x1 = jnp.maximum(dis * (sa_ref[...] + sb_ref[...] + g1_ref[...])
                     + b1_ref[...], 0.0)
    colsum = jnp.sum(x1, axis=0, keepdims=True)

    @pl.when(i == 0)
    def _():
        m1_ref[...] = colsum

    @pl.when(i > 0)
    def _():
        m1_ref[...] += colsum

    h2 = jnp.dot(x1, w2_ref[...], preferred_element_type=jnp.float32)
    g2_ref[...] = dis * h2


def _k6_body(sa_ref, sb_ref, g2_ref, ha_ref, hb_ref, b2_ref, m1_ref,
             linw_ref, linb_ref, fcw_ref, fcb_ref, out_ref, m2_acc):
    i = pl.program_id(0)
    dis = _dis_block(ha_ref[...], hb_ref[...])
    x2 = jnp.maximum(dis * (sa_ref[...] + sb_ref[...] + g2_ref[...])
                     + b2_ref[...], 0.0)
    colsum = jnp.sum(x2, axis=0, keepdims=True)

    @pl.when(i == 0)
    def _():
        m2_acc[...] = colsum

    @pl.when(i > 0)
    def _():
        m2_acc[...] += colsum

    @pl.when(i == _GRID - 1)
    def _():
        inv_n = 1.0 / N
        m1 = m1_ref[...] * inv_n
        m2 = m2_acc[...] * inv_n
        pooled = (jnp.dot(m1, linw_ref[0:D, :], preferred_element_type=jnp.float32)
                  + jnp.dot(m2, linw_ref[D:2 * D, :], preferred_element_type=jnp.float32)
                  + linb_ref[...])
        logits = jnp.dot(pooled, fcw_ref[...], preferred_element_type=jnp.float32) \
            + fcb_ref[...]
        zmax = jnp.max(logits, axis=1, keepdims=True)
        ez = jnp.exp(logits - zmax)
        out_ref[...] = ez / jnp.sum(ez, axis=1, keepdims=True)


def kernel(x, edge_index, W1, b1, W2, b2, lin_W, lin_b, fc_W, fc_b):
    src = edge_index[0].astype(jnp.int32).reshape(NW * NCH, CHUNK)
    dst = edge_index[1].astype(jnp.int32).reshape(NW * NCH, CHUNK)
    e1 = jnp.zeros((CHUNK, HW), jnp.float32).at[:, 0].set(1.0)
    z16 = jnp.zeros((CC, HW), jnp.float32)
    zrow = jnp.zeros((CC, D), jnp.float32)
    b1r = b1.reshape(1, D)
    b2r = b2.reshape(1, D)
    linbr = lin_b.reshape(1, D)
    fcbr = fc_b.reshape(1, DOUT)

    hist = _get_deg_kernel()(dst, e1, z16)

    top = lambda i: (i, 0)
    bot = lambda i: (i + _GRID, 0)
    row_t = pl.BlockSpec((_RB, D), top)
    row_b = pl.BlockSpec((_RB, D), bot)
    h_spec_t = pl.BlockSpec((_RB, HW), top)
    h_spec_b = pl.BlockSpec((_RB, HW), bot)
    w_full = pl.BlockSpec((D, D), lambda i: (0, 0))
    vec_full = pl.BlockSpec((1, D), lambda i: (0, 0))

    g1 = pl.pallas_call(
        _k2_body,
        grid=(_GRID,),
        in_specs=[row_t, w_full, h_spec_t, h_spec_b],
        out_specs=row_t,
        out_shape=jax.ShapeDtypeStruct((N, D), jnp.float32),
    )(x, W1, hist, hist)

    s1 = _get_spmm_kernel()(g1, src, dst, zrow)

    g2, m1sum = pl.pallas_call(
        _k4_body,
        grid=(_GRID,),
        in_specs=[row_t, row_b, row_t, h_spec_t, h_spec_b, vec_full, w_full],
        out_specs=[row_t, vec_full],
        out_shape=[jax.ShapeDtypeStruct((N, D), jnp.float32),
                   jax.ShapeDtypeStruct((1, D), jnp.float32)],
    )(s1, s1, g1, hist, hist, b1r, W2)

    s2 = _get_spmm_kernel()(g2, src, dst, zrow)

    out = pl.pallas_call(
        _k6_body,
        grid=(_GRID,),
        in_specs=[row_t, row_b, row_t, h_spec_t, h_spec_b, vec_full, vec_full,
                  pl.BlockSpec((2 * D, D), lambda i: (0, 0)),
                  vec_full,
                  pl.BlockSpec((D, DOUT), lambda i: (0, 0)),
                  pl.BlockSpec((1, DOUT), lambda i: (0, 0))],
        out_specs=pl.BlockSpec((1, DOUT), lambda i: (0, 0)),
        out_shape=jax.ShapeDtypeStruct((1, DOUT), jnp.float32),
        scratch_shapes=[pltpu.VMEM((1, D), jnp.float32)],
    )(s2, s2, g2, hist, hist, b2r, m1sum, lin_W, linbr, fc_W, fcbr)

    return out
